# Initial kernel scaffold; baseline (speedup 1.0000x reference)
import functools

import jax
import jax.numpy as jnp
from jax import lax
from jax.experimental import pallas as pl
from jax.experimental.pallas import tpu as pltpu
from jax.experimental.pallas import tpu_sc as plsc

H = 16
S = 2048
TBL = 8192
COPY = 4096
FIRE = 16


_mesh = plsc.VectorSubcoreMesh(core_axis_name="c", subcore_axis_name="s")


@functools.partial(
    pl.kernel,
    out_type=jax.ShapeDtypeStruct((H, S, S), jnp.float32),
    mesh=_mesh,
    scratch_types=[
        pltpu.VMEM((TBL,), jnp.float32),
        pltpu.VMEM((8 * COPY,), jnp.float32),
        pltpu.SemaphoreType.DMA,
    ],
)
def _rel_pos_bias(tbl_hbm, out_hbm, t_v, tab_v, sem):
    cid = lax.axis_index("c")
    sid = lax.axis_index("s")
    wid = sid * 2 + cid
    h = wid // 2
    half = wid % 2

    pltpu.sync_copy(tbl_hbm.at[h], t_v)

    def build(k, carry):
        base = 6127 - 16 * k
        for c in range(8):
            v = t_v[pl.ds(base - c, 16)]
            tab_v[pl.ds(c * COPY + 16 * k, 16)] = jnp.flip(v, 0)
        return carry

    lax.fori_loop(0, COPY // 16, build, 0)

    row0 = half * (S // 2)

    def fire(g, carry):
        i0 = row0 + g * FIRE
        copies = []
        for t in range(FIRE):
            i = i0 + t
            rel = (S - 1) - i
            c = lax.bitwise_and(rel, 7)
            src = c * COPY + (rel - c)
            cp = pltpu.make_async_copy(
                tab_v.at[pl.ds(src, S)], out_hbm.at[h, i], sem
            )
            cp.start()
            copies.append(cp)
        for cp in copies:
            cp.wait()
        return carry

    lax.fori_loop(0, (S // 2) // FIRE, fire, 0)


def kernel(rel_bias, seq_len):
    del seq_len
    tbl = jnp.pad(rel_bias[0], ((0, 0), (0, TBL - rel_bias.shape[-1])))
    return _rel_pos_bias(tbl)[None]


# speedup vs baseline: 43.2500x; 43.2500x over previous
import functools

import jax
import jax.numpy as jnp
from jax import lax
from jax.experimental import pallas as pl
from jax.experimental.pallas import tpu as pltpu
from jax.experimental.pallas import tpu_sc as plsc

H = 16
S = 2048
TBL = 8192
COPY = 4096
FIRE = 16


_mesh = plsc.VectorSubcoreMesh(core_axis_name="c", subcore_axis_name="s")


@functools.partial(
    pl.kernel,
    out_type=jax.ShapeDtypeStruct((H * S * S,), jnp.float32),
    mesh=_mesh,
    scratch_types=[
        pltpu.VMEM((TBL,), jnp.float32),
        pltpu.VMEM((8 * COPY,), jnp.float32),
        pltpu.SemaphoreType.DMA,
    ],
)
def _rel_pos_bias(tbl_hbm, out_hbm, t_v, tab_v, sem):
    cid = lax.axis_index("c")
    sid = lax.axis_index("s")
    wid = sid * 2 + cid
    h = wid // 2
    half = wid % 2

    pltpu.sync_copy(tbl_hbm.at[h], t_v)

    lanes = lax.iota(jnp.int32, 16)

    def lane_perm(v, idx):
        dnums = lax.GatherDimensionNumbers(
            offset_dims=(), collapsed_slice_dims=(0,), start_index_map=(0,)
        )
        return lax.gather(
            v, idx[:, None], dnums, (1,),
            mode=lax.GatherScatterMode.PROMISE_IN_BOUNDS,
        )

    def build(k, carry):
        a = pl.multiple_of(6128 - 16 * k, 16)
        va = t_v[pl.ds(a, 16)]
        vp = t_v[pl.ds(a - 16, 16)]
        for c in range(8):
            idx = (15 - c - lanes) & 15
            mask = lanes <= (15 - c)
            chunk = jnp.where(mask, lane_perm(va, idx), lane_perm(vp, idx))
            dst = pl.multiple_of(c * COPY + 16 * k, 16)
            tab_v[pl.ds(dst, 16)] = chunk
        return carry

    lax.fori_loop(0, COPY // 16, build, 0)

    row0 = half * (S // 2)

    def fire(g, carry):
        i0 = row0 + g * FIRE
        copies = []
        for t in range(FIRE):
            i = i0 + t
            rel = (S - 1) - i
            c = lax.bitwise_and(rel, 7)
            src = pl.multiple_of(c * COPY + (rel - c), 8)
            cp = pltpu.make_async_copy(
                tab_v.at[pl.ds(src, S)],
                out_hbm.at[pl.ds((h * S + i) * S, S)],
                sem,
            )
            cp.start()
            copies.append(cp)
        for cp in copies:
            cp.wait()
        return carry

    lax.fori_loop(0, (S // 2) // FIRE, fire, 0)


def kernel(rel_bias, seq_len):
    del seq_len
    tbl = jnp.pad(rel_bias[0], ((0, 0), (1, 0)))
    return _rel_pos_bias(tbl).reshape(1, H, S, S)


# speedup vs baseline: 149.6339x; 3.4597x over previous
import functools

import jax
import jax.numpy as jnp
from jax import lax
from jax.experimental import pallas as pl
from jax.experimental.pallas import tpu as pltpu
from jax.experimental.pallas import tpu_sc as plsc

H = 16
S = 2048
TBL = 8192
COPY = 4096
NCLS = 8


_mesh = plsc.VectorSubcoreMesh(core_axis_name="c", subcore_axis_name="s")


@functools.partial(
    pl.kernel,
    out_type=jax.ShapeDtypeStruct((H, S, S), jnp.float32),
    mesh=_mesh,
    scratch_types=[
        pltpu.VMEM((TBL,), jnp.float32),
        pltpu.VMEM((2, 8, COPY), jnp.float32),
        pltpu.SemaphoreType.DMA,
    ],
)
def _rel_pos_bias(tbl_hbm, out_hbm, t_v, tab_v, sem):
    cid = lax.axis_index("c")
    sid = lax.axis_index("s")
    wid = sid * 2 + cid
    h = wid // 2
    chalf = wid % 2

    pltpu.sync_copy(tbl_hbm.at[h], t_v)

    lanes = lax.iota(jnp.int32, 16)

    def lane_perm(v, idx):
        dnums = lax.GatherDimensionNumbers(
            offset_dims=(), collapsed_slice_dims=(0,), start_index_map=(0,)
        )
        return lax.gather(
            v, idx[:, None], dnums, (1,),
            mode=lax.GatherScatterMode.PROMISE_IN_BOUNDS,
        )

    pending = {0: [], 1: []}

    for u in range(NCLS):
        buf = u % 2
        for cp in pending[buf]:
            cp.wait()
        pending[buf] = []

        dqs = [(7 - r + 8 * u) // 16 for r in range(8)]
        dq_min = min(dqs)

        def build(k, carry, u=u, buf=buf, dqs=dqs, dq_min=dq_min):
            a = pl.multiple_of(6128 - 16 * k - 64 * chalf - 16 * dq_min, 16)
            w = [t_v[pl.ds(a, 16)], t_v[pl.ds(a - 16, 16)]]
            if max(dqs) > dq_min:
                w.append(t_v[pl.ds(a - 32, 16)])
            for r in range(8):
                cs = (7 - r + 8 * u) % 16
                rel = dqs[r] - dq_min
                idx = (15 - cs - lanes) & 15
                mask = lanes <= (15 - cs)
                chunk = jnp.where(
                    mask, lane_perm(w[rel], idx), lane_perm(w[rel + 1], idx)
                )
                tab_v[buf, r, pl.ds(pl.multiple_of(16 * k, 16), 16)] = chunk
            return carry

        lax.fori_loop(0, COPY // 16, build, 0)

        blk_base = 15 - 8 * chalf - u
        for j in range(16):
            blk = blk_base + 16 * j
            f = pl.multiple_of(2040 - 8 * blk - 64 * chalf - 8 * u, 128)
            rs = pl.multiple_of(8 * blk, 8)
            cp = pltpu.make_async_copy(
                tab_v.at[buf, :, pl.ds(f, S)],
                out_hbm.at[h, pl.ds(rs, 8), :],
                sem,
            )
            cp.start()
            pending[buf].append(cp)

    for buf in (0, 1):
        for cp in pending[buf]:
            cp.wait()


def kernel(rel_bias, seq_len):
    del seq_len
    tbl = jnp.pad(rel_bias[0], ((0, 0), (1, 0)))
    return _rel_pos_bias(tbl)[None]


# speedup vs baseline: 153.9135x; 1.0286x over previous
import functools

import jax
import jax.numpy as jnp
from jax import lax
from jax.experimental import pallas as pl
from jax.experimental.pallas import tpu as pltpu
from jax.experimental.pallas import tpu_sc as plsc

H = 16
S = 2048
TBL = 8191
COPY = 4096


_mesh = plsc.VectorSubcoreMesh(core_axis_name="c", subcore_axis_name="s")


@functools.partial(
    pl.kernel,
    out_type=jax.ShapeDtypeStruct((H, S, S), jnp.float32),
    mesh=_mesh,
    scratch_types=[
        pltpu.VMEM((TBL,), jnp.float32),
        pltpu.VMEM((2, 8, COPY), jnp.float32),
        pltpu.SemaphoreType.DMA,
    ],
)
def _rel_pos_bias(tbl_hbm, out_hbm, t_v, tab_v, sem):
    cid = lax.axis_index("c")
    sid = lax.axis_index("s")
    wid = sid * 2 + cid
    h = wid // 2
    chalf = wid % 2

    pltpu.sync_copy(tbl_hbm.at[h], t_v)

    lanes = lax.iota(jnp.int32, 16)

    def lane_perm(v, idx):
        dnums = lax.GatherDimensionNumbers(
            offset_dims=(), collapsed_slice_dims=(0,), start_index_map=(0,)
        )
        return lax.gather(
            v, idx[:, None], dnums, (1,),
            mode=lax.GatherScatterMode.PROMISE_IN_BOUNDS,
        )

    def drain_one(buf):
        pltpu.make_async_copy(
            tab_v.at[buf, :, pl.ds(0, S)],
            out_hbm.at[h, pl.ds(0, 8), :],
            sem,
        ).wait()

    def fire_class(buf, u):
        def fire(j, carry):
            blk = (15 - 8 * chalf - u) + 16 * j
            f = pl.multiple_of(1920 - 128 * j, 128)
            rs = pl.multiple_of(8 * blk, 8)
            pltpu.make_async_copy(
                tab_v.at[buf, :, pl.ds(f, S)],
                out_hbm.at[h, pl.ds(rs, 8), :],
                sem,
            ).start()
            return carry

        lax.fori_loop(0, 16, fire, 0)

    def build_class(buf, parity, up):
        nw = 2 if parity == 0 else 3

        def build(k, carry):
            base = -16 * up - 64 * chalf - 16 * k
            a_top = pl.multiple_of(base + 6128, 16)
            w = [t_v[pl.ds(a_top - 16 * n, 16)] for n in range(nw)]
            for r in range(8):
                cs = (7 + r - 8 * parity) % 16
                ar = 6135 + r - 8 * parity - cs
                rel = (6128 - ar) // 16
                assert ar + 16 * rel == 6128 and rel + 1 < nw, (r, parity)
                idx = (cs - lanes) & 15
                mask = lanes <= cs
                chunk = jnp.where(
                    mask, lane_perm(w[rel], idx), lane_perm(w[rel + 1], idx)
                )
                tab_v[buf, r, pl.ds(pl.multiple_of(16 * k, 16), 16)] = chunk
            return carry

        lax.fori_loop(0, COPY // 16, build, 0)

    def pair(up, carry):
        for parity in range(2):
            buf = parity

            @pl.when(up > 0)
            def _drain(buf=buf):
                def d(j, c):
                    drain_one(buf)
                    return c

                lax.fori_loop(0, 16, d, 0)

            build_class(buf, parity, up)
            fire_class(buf, 2 * up + parity)
        return carry

    lax.fori_loop(0, 4, pair, 0)

    def final_drain(j, carry):
        drain_one(0)
        drain_one(1)
        return carry

    lax.fori_loop(0, 16, final_drain, 0)


def kernel(rel_bias, seq_len):
    del seq_len
    return _rel_pos_bias(rel_bias[0])[None]
